# Initial kernel scaffold; baseline (speedup 1.0000x reference)
import jax
import jax.numpy as jnp
from jax.experimental import pallas as pl


def kernel(x, adj, W1, b1, W2, b2):
    raise NotImplementedError("write your pallas kernel here")


# speedup vs baseline: 3.4909x; 3.4909x over previous
import jax
import jax.numpy as jnp
from jax.experimental import pallas as pl

_N = 2048


def _gcn_body(x_ref, adj_ref, W1_ref, b1_ref, W2_ref, b2_ref, out_ref):
    A = adj_ref[0]
    n = A.shape[0]

    rows = jax.lax.broadcasted_iota(jnp.int32, (n, n), 0)
    cols = jax.lax.broadcasted_iota(jnp.int32, (n, n), 1)
    eye = rows == cols
    diag = jnp.sum(jnp.where(eye, A, 0.0), axis=1)
    c = jnp.where(diag == 0.0, 1.0, 0.0)

    deg = jnp.sum(A, axis=0) + c
    dinv = jax.lax.rsqrt(deg)[:, None]
    cd = (c * jnp.squeeze(dinv, -1))[:, None]

    A_bf = A.astype(jnp.bfloat16)

    def layer(h_in, W, b):
        h = jnp.dot(h_in, W[...], preferred_element_type=jnp.float32)
        v = dinv * h
        agg = jax.lax.dot_general(
            A_bf, v.astype(jnp.bfloat16),
            (((0,), (0,)), ((), ())),
            preferred_element_type=jnp.float32,
        )
        agg = agg + cd * h
        return jnp.maximum(dinv * agg + b[...], 0.0)

    h1 = layer(x_ref[0], W1_ref, b1_ref)
    out_ref[0] = layer(h1, W2_ref, b2_ref)


@jax.jit
def kernel(x, adj, W1, b1, W2, b2):
    B, N, F = x.shape
    H = W2.shape[1]
    out = pl.pallas_call(
        _gcn_body,
        grid=(B,),
        in_specs=[
            pl.BlockSpec((1, N, F), lambda b: (b, 0, 0)),
            pl.BlockSpec((1, N, N), lambda b: (b, 0, 0)),
            pl.BlockSpec((F, H), lambda b: (0, 0)),
            pl.BlockSpec((1, H), lambda b: (0, 0)),
            pl.BlockSpec((H, H), lambda b: (0, 0)),
            pl.BlockSpec((1, H), lambda b: (0, 0)),
        ],
        out_specs=pl.BlockSpec((1, N, H), lambda b: (b, 0, 0)),
        out_shape=jax.ShapeDtypeStruct((B, N, H), jnp.float32),
    )(x, adj, W1, b1.reshape(1, H), W2, b2.reshape(1, H))
    return out
